# Initial kernel scaffold; baseline (speedup 1.0000x reference)
import jax
import jax.numpy as jnp
from jax.experimental import pallas as pl


def kernel(features, edge_index, params):
    raise NotImplementedError("write your pallas kernel here")


# speedup vs baseline: 4.3542x; 4.3542x over previous
import functools

import jax
import jax.numpy as jnp
from jax import lax
from jax.experimental import pallas as pl
from jax.experimental.pallas import tpu as pltpu
from jax.experimental.pallas import tpu_sc as plsc

_N = 10000
_E = 160000
_NTILES = 16
_CK = 80
_EPT = _E // _NTILES
_NCH = _EPT // _CK
_RB = 2000
_ZR = 640


def _dot(a, b):
    return jax.lax.dot_general(
        a.astype(jnp.bfloat16), b.astype(jnp.bfloat16),
        (((1,), (0,)), ((), ())),
        preferred_element_type=jnp.float32)


def _make_agg(d2):
    mesh = plsc.VectorSubcoreMesh(core_axis_name="c", subcore_axis_name="s")

    @functools.partial(
        pl.kernel,
        out_type=(jax.ShapeDtypeStruct((_N, d2), jnp.float32),
                  jax.ShapeDtypeStruct((_N, d2), jnp.float32)),
        mesh=mesh,
        scratch_types=[
            pltpu.VMEM((_EPT,), jnp.int32),
            pltpu.VMEM((_NCH, _CK), jnp.int32),
            pltpu.VMEM((_CK, d2), jnp.float32),
            pltpu.VMEM_SHARED((_N, d2), jnp.float32),
            pltpu.SemaphoreType.DMA,
        ],
        compiler_params=pltpu.CompilerParams(use_tc_tiling_on_sc=False),
    )
    def agg(h0, h1, rows2, cols, zeros, out0, out1,
            cols_v, rows_v, gbuf, acc, gsem):
        c = lax.axis_index("c")
        s = lax.axis_index("s")
        base = jnp.minimum(s * _ZR, _N - _ZR)

        pltpu.sync_copy(zeros.at[pl.ds(base, _ZR)], acc.at[pl.ds(base, _ZR)])
        pltpu.sync_copy(cols.at[pl.ds(s * _EPT, _EPT)], cols_v)
        pltpu.sync_copy(rows2.at[s], rows_v)
        plsc.subcore_barrier()

        def loop(h):
            def body(j, carry):
                off = pl.multiple_of(j * _CK, _CK)
                idx = cols_v.at[pl.ds(off, _CK)]
                pltpu.async_copy(h.at[idx], gbuf, gsem).wait()
                pltpu.sync_copy(gbuf, acc.at[rows_v.at[j]], add=True)
                return carry
            lax.fori_loop(0, _NCH, body, 0)

        pl.when(c == 0)(lambda: loop(h0))
        pl.when(c == 1)(lambda: loop(h1))
        plsc.subcore_barrier()

        def wb(out):
            pltpu.sync_copy(acc.at[pl.ds(base, _ZR)], out.at[pl.ds(base, _ZR)])
        pl.when(c == 0)(lambda: wb(out0))
        pl.when(c == 1)(lambda: wb(out1))

    return agg


def _bn_relu(z, g, be):
    m = jnp.mean(z, axis=0, keepdims=True)
    v = jnp.mean((z - m) ** 2, axis=0, keepdims=True)
    return jnp.maximum((z - m) / jnp.sqrt(v + 1e-5) * g + be, 0.0)


def _matmul_split(h, w, nslabs):
    k = h.shape[1]
    dout = w.shape[1]
    ds_ = dout // nslabs

    def body(h_ref, w_ref, *out_refs):
        z = _dot(h_ref[...], w_ref[...])
        for t, o in enumerate(out_refs):
            o[...] = z[:, t * ds_:(t + 1) * ds_]

    return pl.pallas_call(
        body,
        grid=(_N // _RB,),
        in_specs=[
            pl.BlockSpec((_RB, k), lambda i: (i, 0)),
            pl.BlockSpec((k, dout), lambda i: (0, 0)),
        ],
        out_specs=tuple(pl.BlockSpec((_RB, ds_), lambda i: (i, 0))
                        for _ in range(nslabs)),
        out_shape=tuple(jax.ShapeDtypeStruct((_N, ds_), jnp.float32)
                        for _ in range(nslabs)),
    )(h, w)


def _matmul(h, w):
    k = h.shape[1]
    dout = w.shape[1]

    def body(h_ref, w_ref, out_ref):
        out_ref[...] = _dot(h_ref[...], w_ref[...])

    return pl.pallas_call(
        body,
        grid=(_N // _RB,),
        in_specs=[
            pl.BlockSpec((_RB, k), lambda i: (i, 0)),
            pl.BlockSpec((k, dout), lambda i: (0, 0)),
        ],
        out_specs=pl.BlockSpec((_RB, dout), lambda i: (i, 0)),
        out_shape=jax.ShapeDtypeStruct((_N, dout), jnp.float32),
    )(h, w)


def _bn_merge2(a0, a1, g, be):
    d2 = a0.shape[1]
    dout = 2 * d2

    def body(a0_ref, a1_ref, g_ref, be_ref, out_ref):
        out_ref[:, :d2] = _bn_relu(a0_ref[...], g_ref[:, :d2], be_ref[:, :d2])
        out_ref[:, d2:] = _bn_relu(a1_ref[...], g_ref[:, d2:], be_ref[:, d2:])

    return pl.pallas_call(
        body,
        out_shape=jax.ShapeDtypeStruct((_N, dout), jnp.float32),
    )(a0, a1, g, be)


def _bn_merge_stack(a_stack, g, be):
    nslabs = a_stack.shape[0]
    d2 = a_stack.shape[2]
    dout = nslabs * d2

    def body(a_ref, g_ref, be_ref, out_ref):
        out_ref[...] = _bn_relu(a_ref[0], g_ref[...], be_ref[...])

    return pl.pallas_call(
        body,
        grid=(nslabs,),
        in_specs=[
            pl.BlockSpec((1, _N, d2), lambda i: (i, 0, 0)),
            pl.BlockSpec((1, d2), lambda i: (0, i)),
            pl.BlockSpec((1, d2), lambda i: (0, i)),
        ],
        out_specs=pl.BlockSpec((_N, d2), lambda i: (0, i)),
        out_shape=jax.ShapeDtypeStruct((_N, dout), jnp.float32),
    )(a_stack, g, be)


def _bn_relu_call(z, g, be):
    dout = z.shape[1]
    bw = min(dout, 128)
    nb = dout // bw

    def body(z_ref, g_ref, be_ref, out_ref):
        out_ref[...] = _bn_relu(z_ref[...], g_ref[...], be_ref[...])

    return pl.pallas_call(
        body,
        grid=(nb,),
        in_specs=[
            pl.BlockSpec((_N, bw), lambda i: (0, i)),
            pl.BlockSpec((1, bw), lambda i: (0, i)),
            pl.BlockSpec((1, bw), lambda i: (0, i)),
        ],
        out_specs=pl.BlockSpec((_N, bw), lambda i: (0, i)),
        out_shape=jax.ShapeDtypeStruct((_N, dout), jnp.float32),
    )(z, g, be)


def _head(h, fc, fc4):
    r = lambda a: a.reshape(1, -1)
    for p in fc:
        h = _bn_relu_call(_matmul(h, p['W']), r(p['g']), r(p['be']))

    def out_body(h_ref, w, b, out_ref):
        out_ref[...] = _dot(h_ref[...], w[...]) + b[...]

    return pl.pallas_call(
        out_body, out_shape=jax.ShapeDtypeStruct((_N, 5), jnp.float32),
    )(h, fc4['W'], r(fc4['b']))


def kernel(features, edge_index, params):
    rows = edge_index[0]
    cols = edge_index[1]
    rows2 = rows.reshape(_NTILES, _NCH, _CK)

    gcn = params['gcn']
    zeros = jnp.zeros((_N, 128), jnp.float32)
    r = lambda a: a.reshape(1, -1)

    h = features
    for l in range(6):
        w = gcn[l]['W']
        dout = w.shape[1]
        g, be = r(gcn[l]['g']), r(gcn[l]['be'])
        if dout <= 256:
            d2 = dout // 2
            z0, z1 = _matmul_split(h, w, 2)
            a0, a1 = _make_agg(d2)(z0, z1, rows2, cols, zeros[:, :d2])
            h = _bn_merge2(a0, a1, g, be)
        else:
            z0, z1, z2, z3 = _matmul_split(h, w, 4)
            agg = _make_agg(128)
            a0, a1 = agg(z0, z1, rows2, cols, zeros)
            a2, a3 = agg(z2, z3, rows2, cols, zeros)
            h = _bn_merge_stack(jnp.stack([a0, a1, a2, a3]), g, be)

    return _head(h, params['fc'], params['fc4'])
